# Initial kernel scaffold; baseline (speedup 1.0000x reference)
import jax
import jax.numpy as jnp
from jax.experimental import pallas as pl


def kernel(x, idxs, W_fe, b_fe, W_a1, b_a1, W_a2, b_a2, W_p, b_p):
    raise NotImplementedError("write your pallas kernel here")


# speedup vs baseline: 4.3799x; 4.3799x over previous
import jax
import jax.numpy as jnp
from jax import lax
from jax.experimental import pallas as pl
from jax.experimental.pallas import tpu as pltpu

NSEG = 16
BLK = 2048


def _body(seg_ref, x_ref, wfe_ref, bfe_ref, wa1_ref, ba1_ref, wa2_ref,
          wp_ref, bp_ref, m_out_ref, p_out_ref, macc, dacc, mmax):
    i = pl.program_id(0)
    nb = pl.num_programs(0)
    neg = jnp.float32(-1e30)

    @pl.when(i == 0)
    def _init():
        macc[...] = jnp.zeros_like(macc)
        dacc[...] = jnp.zeros_like(dacc)
        mmax[...] = jnp.full_like(mmax, neg)

    x = x_ref[...]
    h = jnp.dot(x, wfe_ref[...],
                preferred_element_type=jnp.float32) + bfe_ref[...]
    t = jnp.tanh(jnp.dot(h, wa1_ref[...],
                         preferred_element_type=jnp.float32) + ba1_ref[...])
    a = jnp.dot(t, wa2_ref[...],
                preferred_element_type=jnp.float32)

    seg = seg_ref[...]
    lane = lax.broadcasted_iota(jnp.int32, (1, NSEG), 1).astype(jnp.float32)
    oh = seg == lane

    mblk = jnp.max(jnp.where(oh, a, neg), axis=0, keepdims=True)
    mold = mmax[...]
    mnew = jnp.maximum(mold, mblk)
    scale = jnp.exp(mold - mnew)
    e = jnp.where(oh, jnp.exp(a - mnew), 0.0)
    mmax[...] = mnew
    dacc[...] = dacc[...] * scale + jnp.sum(e, axis=0, keepdims=True)
    macc[...] = macc[...] * scale + lax.dot_general(
        h, e, (((0,), (0,)), ((), ())),
        preferred_element_type=jnp.float32)

    @pl.when(i == nb - 1)
    def _fin():
        d = jnp.maximum(dacc[...], jnp.float32(1e-30))
        mt = macc[...] / d
        m_out_ref[...] = mt.T
        proj = lax.dot_general(mt, wp_ref[...], (((0,), (0,)), ((), ())),
                               preferred_element_type=jnp.float32) + bp_ref[...]
        n2 = jnp.sum(proj * proj, axis=1, keepdims=True)
        p_out_ref[...] = proj / jnp.maximum(jnp.sqrt(n2), jnp.float32(1e-12))


def kernel(x, idxs, W_fe, b_fe, W_a1, b_a1, W_a2, b_a2, W_p, b_p):
    n, l = x.shape[1], x.shape[2]
    d, f = W_fe.shape[0], W_a1.shape[0]
    nb = n // BLK

    xs = x.reshape(n, l)
    segf = idxs.astype(jnp.float32).reshape(n, 1)
    wfe = W_fe.T
    bfe = b_fe.reshape(1, d)
    wa1 = W_a1.T
    ba1 = b_a1.reshape(1, f)
    wa2 = W_a2.T
    wp = W_p.T
    bp = b_p.reshape(1, f)

    m_out, p_out = pl.pallas_call(
        _body,
        grid=(nb,),
        in_specs=[
            pl.BlockSpec((BLK, 1), lambda i: (i, 0)),
            pl.BlockSpec((BLK, l), lambda i: (i, 0)),
            pl.BlockSpec((l, d), lambda i: (0, 0)),
            pl.BlockSpec((1, d), lambda i: (0, 0)),
            pl.BlockSpec((d, f), lambda i: (0, 0)),
            pl.BlockSpec((1, f), lambda i: (0, 0)),
            pl.BlockSpec((f, 1), lambda i: (0, 0)),
            pl.BlockSpec((d, f), lambda i: (0, 0)),
            pl.BlockSpec((1, f), lambda i: (0, 0)),
        ],
        out_specs=[
            pl.BlockSpec((NSEG, d), lambda i: (0, 0)),
            pl.BlockSpec((NSEG, f), lambda i: (0, 0)),
        ],
        out_shape=[
            jax.ShapeDtypeStruct((NSEG, d), jnp.float32),
            jax.ShapeDtypeStruct((NSEG, f), jnp.float32),
        ],
        scratch_shapes=[
            pltpu.VMEM((d, NSEG), jnp.float32),
            pltpu.VMEM((1, NSEG), jnp.float32),
            pltpu.VMEM((1, NSEG), jnp.float32),
        ],
        compiler_params=pltpu.CompilerParams(
            dimension_semantics=("arbitrary",),
        ),
    )(segf, xs, wfe, bfe, wa1, ba1, wa2, wp, bp)
    return (m_out, p_out)


# speedup vs baseline: 4.7312x; 1.0802x over previous
import jax
import jax.numpy as jnp
from jax import lax
from jax.experimental import pallas as pl
from jax.experimental.pallas import tpu as pltpu

NSEG = 16
BLK = 4096


def _body(seg_ref, x_ref, wfe_ref, bfe_ref, wa1_ref, ba1_ref, wa2_ref,
          wp_ref, bp_ref, m_out_ref, p_out_ref, macc, dacc, mmax):
    i = pl.program_id(0)
    nb = pl.num_programs(0)
    neg = jnp.float32(-1e30)

    @pl.when(i == 0)
    def _init():
        macc[...] = jnp.zeros_like(macc)
        dacc[...] = jnp.zeros_like(dacc)
        mmax[...] = jnp.full_like(mmax, neg)

    x = x_ref[...]
    h = jnp.dot(x, wfe_ref[...],
                preferred_element_type=jnp.float32) + bfe_ref[...]
    t = jnp.tanh(jnp.dot(h, wa1_ref[...],
                         preferred_element_type=jnp.float32) + ba1_ref[...])
    a = jnp.dot(t, wa2_ref[...],
                preferred_element_type=jnp.float32)

    seg = seg_ref[...]
    lane = lax.broadcasted_iota(jnp.int32, (1, NSEG), 1).astype(jnp.float32)
    oh = seg == lane

    mblk = jnp.max(jnp.where(oh, a, neg), axis=0, keepdims=True)
    mold = mmax[...]
    mnew = jnp.maximum(mold, mblk)
    scale = jnp.exp(mold - mnew)
    e = jnp.where(oh, jnp.exp(a - mnew), 0.0)
    mmax[...] = mnew
    dacc[...] = dacc[...] * scale + jnp.sum(e, axis=0, keepdims=True)
    macc[...] = macc[...] * scale + lax.dot_general(
        h, e, (((0,), (0,)), ((), ())),
        preferred_element_type=jnp.float32)

    @pl.when(i == nb - 1)
    def _fin():
        d = jnp.maximum(dacc[...], jnp.float32(1e-30))
        mt = macc[...] / d
        m_out_ref[...] = mt.T
        proj = lax.dot_general(mt, wp_ref[...], (((0,), (0,)), ((), ())),
                               preferred_element_type=jnp.float32) + bp_ref[...]
        n2 = jnp.sum(proj * proj, axis=1, keepdims=True)
        p_out_ref[...] = proj / jnp.maximum(jnp.sqrt(n2), jnp.float32(1e-12))


def kernel(x, idxs, W_fe, b_fe, W_a1, b_a1, W_a2, b_a2, W_p, b_p):
    n, l = x.shape[1], x.shape[2]
    d, f = W_fe.shape[0], W_a1.shape[0]
    nb = n // BLK

    xs = x.reshape(n, l)
    segf = idxs.astype(jnp.float32).reshape(n, 1)
    wfe = W_fe.T
    bfe = b_fe.reshape(1, d)
    wa1 = W_a1.T
    ba1 = b_a1.reshape(1, f)
    wa2 = W_a2.T
    wp = W_p.T
    bp = b_p.reshape(1, f)

    m_out, p_out = pl.pallas_call(
        _body,
        grid=(nb,),
        in_specs=[
            pl.BlockSpec((BLK, 1), lambda i: (i, 0)),
            pl.BlockSpec((BLK, l), lambda i: (i, 0)),
            pl.BlockSpec((l, d), lambda i: (0, 0)),
            pl.BlockSpec((1, d), lambda i: (0, 0)),
            pl.BlockSpec((d, f), lambda i: (0, 0)),
            pl.BlockSpec((1, f), lambda i: (0, 0)),
            pl.BlockSpec((f, 1), lambda i: (0, 0)),
            pl.BlockSpec((d, f), lambda i: (0, 0)),
            pl.BlockSpec((1, f), lambda i: (0, 0)),
        ],
        out_specs=[
            pl.BlockSpec((NSEG, d), lambda i: (0, 0)),
            pl.BlockSpec((NSEG, f), lambda i: (0, 0)),
        ],
        out_shape=[
            jax.ShapeDtypeStruct((NSEG, d), jnp.float32),
            jax.ShapeDtypeStruct((NSEG, f), jnp.float32),
        ],
        scratch_shapes=[
            pltpu.VMEM((d, NSEG), jnp.float32),
            pltpu.VMEM((1, NSEG), jnp.float32),
            pltpu.VMEM((1, NSEG), jnp.float32),
        ],
        compiler_params=pltpu.CompilerParams(
            dimension_semantics=("arbitrary",),
        ),
    )(segf, xs, wfe, bfe, wa1, ba1, wa2, wp, bp)
    return (m_out, p_out)
